# Initial kernel scaffold; baseline (speedup 1.0000x reference)
import jax
import jax.numpy as jnp
from jax.experimental import pallas as pl


def kernel(node_embed, node_pos, edge_index, edge_attr, A_w, A_b, B_w, W_shared, W_res, W_mlp, b_mlp):
    raise NotImplementedError("write your pallas kernel here")


# speedup vs baseline: 5.1904x; 5.1904x over previous
import functools

import jax
import jax.numpy as jnp
from jax import lax
from jax.experimental import pallas as pl
from jax.experimental.pallas import tpu as pltpu
from jax.experimental.pallas import tpu_sc as plsc

N = 10000
D = 128
H = 128
R = 16
T = 4

NC = 2
NS = 16
NW = NC * NS
CHUNK = 128

SROWS = 10240
UROWS = 4 * SROWS


def _y1_body(x_ref, b_ref, o_ref):
    o_ref[0] = jnp.dot(x_ref[...], b_ref[0].T,
                       preferred_element_type=jnp.float32)


def _build_y1(x, b_src):
    return pl.pallas_call(
        _y1_body,
        grid=(T,),
        in_specs=[
            pl.BlockSpec((N, D), lambda t: (0, 0)),
            pl.BlockSpec((1, R, D), lambda t: (t, 0, 0)),
        ],
        out_specs=pl.BlockSpec((1, N, R), lambda t: (t, 0, 0)),
        out_shape=jax.ShapeDtypeStruct((T, N, R), jnp.float32),
    )(x, b_src)


_SC_MESH = dict(core_axis_name="c", subcore_axis_name="s")


def _sc_s_kernel(ep_per_tile):
    n_chunks = ep_per_tile // CHUNK

    @functools.partial(
        pl.kernel,
        mesh=plsc.VectorSubcoreMesh(**_SC_MESH),
        compiler_params=pltpu.CompilerParams(use_tc_tiling_on_sc=False),
        out_type=[
            jax.ShapeDtypeStruct((NC, SROWS, D), jnp.float32),
        ],
        scratch_types=[
            pltpu.VMEM((CHUNK,), jnp.int32),
            pltpu.VMEM((CHUNK,), jnp.int32),
            pltpu.VMEM((CHUNK, D), jnp.float32),
            pltpu.VMEM_SHARED((SROWS, D), jnp.float32),
            pltpu.SemaphoreType.DMA,
        ],
    )
    def k(x_hbm, row_h, col_h, s_out, row_v, col_v, xbuf, s_sh, sem_x):
        cid = lax.axis_index("c")
        sid = lax.axis_index("s")
        gid = cid * NS + sid

        zx = jnp.zeros((16,), jnp.float32)

        def zxb(i, _):
            def zrow(j, _):
                xbuf[i, pl.ds(j * 16, 16)] = zx
                return 0
            lax.fori_loop(0, D // 16, zrow, 0)
            return 0
        lax.fori_loop(0, CHUNK, zxb, 0)

        s_per_tile = SROWS // NS
        def zs(i, _):
            base = sid * s_per_tile + i * CHUNK
            pltpu.sync_copy(xbuf, s_sh.at[pl.ds(base, CHUNK)])
            return 0
        lax.fori_loop(0, s_per_tile // CHUNK, zs, 0)

        plsc.subcore_barrier()

        edge_base = gid * ep_per_tile

        def body(i, _):
            base = edge_base + i * CHUNK
            pltpu.sync_copy(row_h.at[pl.ds(base, CHUNK)], row_v)
            pltpu.sync_copy(col_h.at[pl.ds(base, CHUNK)], col_v)
            pltpu.async_copy(x_hbm.at[row_v], xbuf, sem_x).wait()
            pltpu.sync_copy(xbuf, s_sh.at[col_v], add=True)
            return 0

        lax.fori_loop(0, n_chunks, body, 0)

        plsc.subcore_barrier()

        def ws(i, _):
            base = sid * s_per_tile + i * CHUNK
            pltpu.sync_copy(s_sh.at[pl.ds(base, CHUNK)],
                            s_out.at[cid, pl.ds(base, CHUNK)])
            return 0
        lax.fori_loop(0, s_per_tile // CHUNK, ws, 0)

    return k


def _sc_u_kernel(ep_per_tile):
    n_chunks = ep_per_tile // CHUNK

    @functools.partial(
        pl.kernel,
        mesh=plsc.VectorSubcoreMesh(**_SC_MESH),
        compiler_params=pltpu.CompilerParams(use_tc_tiling_on_sc=False),
        out_type=[
            jax.ShapeDtypeStruct((NC, UROWS, R), jnp.float32),
            jax.ShapeDtypeStruct((NC, UROWS), jnp.float32),
            jax.ShapeDtypeStruct((NC, UROWS), jnp.float32),
        ],
        scratch_types=[
            pltpu.VMEM((CHUNK,), jnp.int32),
            pltpu.VMEM((CHUNK,), jnp.int32),
            pltpu.VMEM((CHUNK,), jnp.float32),
            pltpu.VMEM((CHUNK,), jnp.float32),
            pltpu.VMEM((CHUNK, R), jnp.float32),
            pltpu.VMEM_SHARED((UROWS, R), jnp.float32),
            pltpu.VMEM_SHARED((UROWS,), jnp.float32),
            pltpu.VMEM_SHARED((UROWS,), jnp.float32),
            pltpu.SemaphoreType.DMA,
        ],
    )
    def k(y1_hbm, gsrc_h, gdst_h, dist_h, u_out, cnt_out, dsum_out,
          gsrc_v, gdst_v, dist_v, ones_v, ybuf,
          u_sh, cnt_sh, dsum_sh, sem_y):
        cid = lax.axis_index("c")
        sid = lax.axis_index("s")
        gid = cid * NS + sid

        zx = jnp.zeros((16,), jnp.float32)
        ox = jnp.ones((16,), jnp.float32)

        def z16(i, _):
            dist_v[pl.ds(i * 16, 16)] = zx
            ones_v[pl.ds(i * 16, 16)] = ox
            return 0
        lax.fori_loop(0, CHUNK // 16, z16, 0)

        def zyb(i, _):
            ybuf[i, pl.ds(0, 16)] = zx
            return 0
        lax.fori_loop(0, CHUNK, zyb, 0)

        u_per_tile = UROWS // NS
        def zu(i, _):
            base = sid * u_per_tile + i * CHUNK
            pltpu.sync_copy(ybuf, u_sh.at[pl.ds(base, CHUNK)])
            pltpu.sync_copy(dist_v, cnt_sh.at[pl.ds(base, CHUNK)])
            pltpu.sync_copy(dist_v, dsum_sh.at[pl.ds(base, CHUNK)])
            return 0
        lax.fori_loop(0, u_per_tile // CHUNK, zu, 0)

        plsc.subcore_barrier()

        edge_base = gid * ep_per_tile

        def body(i, _):
            base = edge_base + i * CHUNK
            pltpu.sync_copy(gsrc_h.at[pl.ds(base, CHUNK)], gsrc_v)
            pltpu.sync_copy(gdst_h.at[pl.ds(base, CHUNK)], gdst_v)
            pltpu.sync_copy(dist_h.at[pl.ds(base, CHUNK)], dist_v)
            pltpu.async_copy(y1_hbm.at[gsrc_v], ybuf, sem_y).wait()
            pltpu.sync_copy(ybuf, u_sh.at[gdst_v], add=True)
            pltpu.sync_copy(ones_v, cnt_sh.at[gdst_v], add=True)
            pltpu.sync_copy(dist_v, dsum_sh.at[gdst_v], add=True)
            return 0

        lax.fori_loop(0, n_chunks, body, 0)

        plsc.subcore_barrier()

        def wu(i, _):
            base = sid * u_per_tile + i * CHUNK
            pltpu.sync_copy(u_sh.at[pl.ds(base, CHUNK)],
                            u_out.at[cid, pl.ds(base, CHUNK)])
            return 0
        lax.fori_loop(0, u_per_tile // CHUNK, wu, 0)
        ubase = sid * u_per_tile
        pltpu.sync_copy(cnt_sh.at[pl.ds(ubase, u_per_tile)],
                        cnt_out.at[cid, pl.ds(ubase, u_per_tile)])
        pltpu.sync_copy(dsum_sh.at[pl.ds(ubase, u_per_tile)],
                        dsum_out.at[cid, pl.ds(ubase, u_per_tile)])

    return k


def _finish_body(x_ref, s_ref, u_ref, cnt_ref, dsum_ref,
                 wsrc_ref, wtgt_ref, wsd_ref, aw_ref, ab_ref,
                 btgt_ref, bd_ref, wres_ref, wmx_ref, wma_ref, bmlp_ref,
                 o_ref):
    xb = x_ref[...]
    S = s_ref[0] + s_ref[1]
    cnt = jnp.sum(cnt_ref[...], axis=0)
    dsum = jnp.sum(dsum_ref[...], axis=0)
    deg = jnp.sum(cnt, axis=0)[:, None]
    dtot = jnp.sum(dsum, axis=0)[:, None]

    low = jnp.zeros_like(xb)
    for t in range(T):
        u_t = u_ref[0, t] + u_ref[1, t]
        y2_t = jnp.dot(xb, btgt_ref[t].T,
                       preferred_element_type=jnp.float32)
        v_t = u_t + cnt[t][:, None] * y2_t + dsum[t][:, None] * bd_ref[t][None, :]
        low = low + jnp.dot(v_t, aw_ref[t].T,
                            preferred_element_type=jnp.float32)
        low = low + cnt[t][:, None] * ab_ref[t][None, :]

    sh = jnp.dot(S, wsrc_ref[...].T, preferred_element_type=jnp.float32)
    sh = sh + deg * jnp.dot(xb, wtgt_ref[...].T,
                            preferred_element_type=jnp.float32)
    sh = sh + dtot * wsd_ref[...][None, :]

    aggr = (low + sh) / jnp.maximum(deg, 1.0)

    out = jnp.dot(xb, wres_ref[...].T, preferred_element_type=jnp.float32)
    pre = (jnp.dot(xb, wmx_ref[...].T, preferred_element_type=jnp.float32)
           + jnp.dot(aggr, wma_ref[...].T, preferred_element_type=jnp.float32)
           + bmlp_ref[...][None, :])
    o_ref[...] = out + jnp.maximum(pre, 0.0)


def _finish(x, s2, u2, cnt2, dsum2, wsrc, wtgt, wsd, a_w, a_b,
            b_tgt, b_d, w_res, wm_x, wm_a, b_mlp):
    BN = 1024
    grid = (SROWS // BN,)
    return pl.pallas_call(
        _finish_body,
        grid=grid,
        in_specs=[
            pl.BlockSpec((BN, D), lambda i: (i, 0)),
            pl.BlockSpec((NC, BN, D), lambda i: (0, i, 0)),
            pl.BlockSpec((NC, T, BN, R), lambda i: (0, 0, i, 0)),
            pl.BlockSpec((NC, T, BN), lambda i: (0, 0, i)),
            pl.BlockSpec((NC, T, BN), lambda i: (0, 0, i)),
            pl.BlockSpec((H, D), lambda i: (0, 0)),
            pl.BlockSpec((H, D), lambda i: (0, 0)),
            pl.BlockSpec((H,), lambda i: (0,)),
            pl.BlockSpec((T, H, R), lambda i: (0, 0, 0)),
            pl.BlockSpec((T, H), lambda i: (0, 0)),
            pl.BlockSpec((T, R, D), lambda i: (0, 0, 0)),
            pl.BlockSpec((T, R), lambda i: (0, 0)),
            pl.BlockSpec((H, D), lambda i: (0, 0)),
            pl.BlockSpec((H, D), lambda i: (0, 0)),
            pl.BlockSpec((H, H), lambda i: (0, 0)),
            pl.BlockSpec((H,), lambda i: (0,)),
        ],
        out_specs=pl.BlockSpec((BN, H), lambda i: (i, 0)),
        out_shape=jax.ShapeDtypeStruct((SROWS, H), jnp.float32),
    )(x, s2, u2, cnt2, dsum2, wsrc, wtgt, wsd, a_w, a_b,
      b_tgt, b_d, w_res, wm_x, wm_a, b_mlp)


def kernel(node_embed, node_pos, edge_index, edge_attr,
           A_w, A_b, B_w, W_shared, W_res, W_mlp, b_mlp):
    E = edge_index.shape[1]
    row = edge_index[0].astype(jnp.int32)
    col = edge_index[1].astype(jnp.int32)
    etype = edge_attr[:, 0].astype(jnp.int32)
    dist = edge_attr[:, 1].astype(jnp.float32)

    wsrc = W_shared[:, :D]
    wtgt = W_shared[:, D:2 * D]
    wsd = W_shared[:, 2 * D]
    b_src = B_w[:, :, :D]
    b_tgt = B_w[:, :, D:2 * D]
    b_d = B_w[:, :, 2 * D]
    wm_x = W_mlp[:, :D]
    wm_a = W_mlp[:, D:]

    ep = ((E + NW * CHUNK - 1) // (NW * CHUNK)) * (NW * CHUNK)
    pad = ep - E
    gsrc = etype * N + row
    gdst = etype * SROWS + col
    row_p = jnp.concatenate([row, jnp.zeros((pad,), jnp.int32)])
    col_p = jnp.concatenate([col, jnp.full((pad,), N, jnp.int32)])
    gsrc_p = jnp.concatenate([gsrc, jnp.zeros((pad,), jnp.int32)])
    gdst_p = jnp.concatenate([gdst, jnp.full((pad,), UROWS - 1, jnp.int32)])
    dist_p = jnp.concatenate([dist, jnp.zeros((pad,), jnp.float32)])

    y1 = _build_y1(node_embed, b_src).reshape(T * N, R)

    (s2,) = _sc_s_kernel(ep // NW)(node_embed, row_p, col_p)
    u2, cnt2, dsum2 = _sc_u_kernel(ep // NW)(y1, gsrc_p, gdst_p, dist_p)

    u2 = u2.reshape(NC, T, SROWS, R)
    cnt2 = cnt2.reshape(NC, T, SROWS)
    dsum2 = dsum2.reshape(NC, T, SROWS)

    x_pad = jnp.concatenate(
        [node_embed, jnp.zeros((SROWS - N, D), jnp.float32)])
    out = _finish(x_pad, s2, u2, cnt2, dsum2, wsrc, wtgt, wsd,
                  A_w, A_b, b_tgt, b_d, W_res, wm_x, wm_a, b_mlp)
    return out[:N]


# speedup vs baseline: 6.2835x; 1.2106x over previous
import functools

import jax
import jax.numpy as jnp
from jax import lax
from jax.experimental import pallas as pl
from jax.experimental.pallas import tpu as pltpu
from jax.experimental.pallas import tpu_sc as plsc

N = 10000
D = 128
H = 128
R = 16
T = 4

NC = 2
NS = 16
NW = NC * NS
CHUNK = 128
CHUNK_S = 96
SCOPY = 80

SROWS = 10240
UROWS = 4 * SROWS


YW = 24


def _y1_body(x_ref, b_ref, o_ref):
    y = jnp.dot(x_ref[...], b_ref[0].T, preferred_element_type=jnp.float32)
    one = jnp.ones((N, 1), jnp.float32)
    zero = jnp.zeros((N, YW - R - 1), jnp.float32)
    o_ref[0] = jnp.concatenate([y, one, zero], axis=-1)


def _build_y1(x, b_src):
    return pl.pallas_call(
        _y1_body,
        grid=(T,),
        in_specs=[
            pl.BlockSpec((N, D), lambda t: (0, 0)),
            pl.BlockSpec((1, R, D), lambda t: (t, 0, 0)),
        ],
        out_specs=pl.BlockSpec((1, N, YW), lambda t: (t, 0, 0)),
        out_shape=jax.ShapeDtypeStruct((T, N, YW), jnp.float32),
    )(x, b_src)


_SC_MESH = dict(core_axis_name="c", subcore_axis_name="s")


def _sc_s_kernel(n_chunks):
    n_pairs = n_chunks // 2

    @functools.partial(
        pl.kernel,
        mesh=plsc.VectorSubcoreMesh(**_SC_MESH),
        compiler_params=pltpu.CompilerParams(use_tc_tiling_on_sc=False),
        out_type=[
            jax.ShapeDtypeStruct((NC, SROWS, D), jnp.float32),
        ],
        scratch_types=[
            pltpu.VMEM((n_chunks, CHUNK_S), jnp.int32),
            pltpu.VMEM((n_chunks, CHUNK_S), jnp.int32),
            pltpu.VMEM((CHUNK_S, D), jnp.float32),
            pltpu.VMEM((CHUNK_S, D), jnp.float32),
            pltpu.VMEM_SHARED((SROWS, D), jnp.float32),
            pltpu.SemaphoreType.DMA,
            pltpu.SemaphoreType.DMA,
        ],
    )
    def k(x_hbm, row_h, col_h, s_out, row3, col3, xb0, xb1, s_sh, sm0, sm1):
        cid = lax.axis_index("c")
        sid = lax.axis_index("s")
        gid = cid * NS + sid

        pltpu.sync_copy(row_h.at[gid], row3)
        pltpu.sync_copy(col_h.at[gid], col3)

        zx = jnp.zeros((16,), jnp.float32)
        def zxb(i, _):
            def zrow(j, _):
                xb0[i, pl.ds(j * 16, 16)] = zx
                return 0
            lax.fori_loop(0, D // 16, zrow, 0)
            return 0
        lax.fori_loop(0, CHUNK_S, zxb, 0)

        s_per_tile = SROWS // NS
        def zs(i, _):
            base = sid * s_per_tile + i * SCOPY
            pltpu.sync_copy(xb0.at[pl.ds(0, SCOPY)],
                            s_sh.at[pl.ds(base, SCOPY)])
            return 0
        lax.fori_loop(0, s_per_tile // SCOPY, zs, 0)

        plsc.subcore_barrier()

        pltpu.async_copy(x_hbm.at[row3.at[0]], xb0, sm0)

        def pair(p, _):
            c0 = 2 * p
            pltpu.async_copy(x_hbm.at[row3.at[c0 + 1]], xb1, sm1)
            pltpu.make_async_copy(x_hbm.at[row3.at[c0]], xb0, sm0).wait()
            pltpu.sync_copy(xb0, s_sh.at[col3.at[c0]], add=True)

            @pl.when(p < n_pairs - 1)
            def _():
                pltpu.async_copy(x_hbm.at[row3.at[c0 + 2]], xb0, sm0)
            pltpu.make_async_copy(x_hbm.at[row3.at[c0 + 1]], xb1, sm1).wait()
            pltpu.sync_copy(xb1, s_sh.at[col3.at[c0 + 1]], add=True)
            return 0

        lax.fori_loop(0, n_pairs, pair, 0)

        plsc.subcore_barrier()

        def ws(i, _):
            base = sid * s_per_tile + i * SCOPY
            pltpu.sync_copy(s_sh.at[pl.ds(base, SCOPY)],
                            s_out.at[cid, pl.ds(base, SCOPY)])
            return 0
        lax.fori_loop(0, s_per_tile // SCOPY, ws, 0)

    return k


def _sc_u_kernel(n_chunks):
    n_pairs = n_chunks // 2

    @functools.partial(
        pl.kernel,
        mesh=plsc.VectorSubcoreMesh(**_SC_MESH),
        compiler_params=pltpu.CompilerParams(use_tc_tiling_on_sc=False),
        out_type=[
            jax.ShapeDtypeStruct((NC, UROWS, YW), jnp.float32),
            jax.ShapeDtypeStruct((NC, UROWS), jnp.float32),
        ],
        scratch_types=[
            pltpu.VMEM((n_chunks, CHUNK), jnp.int32),
            pltpu.VMEM((n_chunks, CHUNK), jnp.int32),
            pltpu.VMEM((n_chunks, CHUNK), jnp.float32),
            pltpu.VMEM((CHUNK, YW), jnp.float32),
            pltpu.VMEM((CHUNK, YW), jnp.float32),
            pltpu.VMEM_SHARED((UROWS, YW), jnp.float32),
            pltpu.VMEM_SHARED((UROWS,), jnp.float32),
            pltpu.SemaphoreType.DMA,
            pltpu.SemaphoreType.DMA,
        ],
    )
    def k(y1_hbm, gsrc_h, gdst_h, dist_h, u_out, dsum_out,
          gsrc3, gdst3, dist3, yb0, yb1, u_sh, dsum_sh, sm0, sm1):
        cid = lax.axis_index("c")
        sid = lax.axis_index("s")
        gid = cid * NS + sid

        pltpu.sync_copy(gsrc_h.at[gid], gsrc3)
        pltpu.sync_copy(gdst_h.at[gid], gdst3)
        pltpu.sync_copy(dist_h.at[gid], dist3)

        zx = jnp.zeros((16,), jnp.float32)
        def zyb(i, _):
            yb0[i, pl.ds(0, 16)] = zx
            yb0[i, pl.ds(YW - 16, 16)] = zx
            return 0
        lax.fori_loop(0, CHUNK, zyb, 0)
        def zd(j, _):
            dist3[0, pl.ds(j * 16, 16)] = zx
            return 0
        lax.fori_loop(0, CHUNK // 16, zd, 0)

        u_per_tile = UROWS // NS
        def zu(i, _):
            base = sid * u_per_tile + i * CHUNK
            pltpu.sync_copy(yb0, u_sh.at[pl.ds(base, CHUNK)])
            pltpu.sync_copy(dist3.at[0], dsum_sh.at[pl.ds(base, CHUNK)])
            return 0
        lax.fori_loop(0, u_per_tile // CHUNK, zu, 0)

        plsc.subcore_barrier()

        pltpu.sync_copy(dist_h.at[gid], dist3)
        pltpu.async_copy(y1_hbm.at[gsrc3.at[0]], yb0, sm0)

        def pair(p, _):
            c0 = 2 * p
            pltpu.async_copy(y1_hbm.at[gsrc3.at[c0 + 1]], yb1, sm1)
            pltpu.make_async_copy(y1_hbm.at[gsrc3.at[c0]], yb0, sm0).wait()
            pltpu.sync_copy(yb0, u_sh.at[gdst3.at[c0]], add=True)
            pltpu.sync_copy(dist3.at[c0], dsum_sh.at[gdst3.at[c0]], add=True)

            @pl.when(p < n_pairs - 1)
            def _():
                pltpu.async_copy(y1_hbm.at[gsrc3.at[c0 + 2]], yb0, sm0)
            pltpu.make_async_copy(y1_hbm.at[gsrc3.at[c0 + 1]], yb1, sm1).wait()
            pltpu.sync_copy(yb1, u_sh.at[gdst3.at[c0 + 1]], add=True)
            pltpu.sync_copy(dist3.at[c0 + 1],
                            dsum_sh.at[gdst3.at[c0 + 1]], add=True)
            return 0

        lax.fori_loop(0, n_pairs, pair, 0)

        plsc.subcore_barrier()

        def wu(i, _):
            base = sid * u_per_tile + i * CHUNK
            pltpu.sync_copy(u_sh.at[pl.ds(base, CHUNK)],
                            u_out.at[cid, pl.ds(base, CHUNK)])
            return 0
        lax.fori_loop(0, u_per_tile // CHUNK, wu, 0)
        ubase = sid * u_per_tile
        pltpu.sync_copy(dsum_sh.at[pl.ds(ubase, u_per_tile)],
                        dsum_out.at[cid, pl.ds(ubase, u_per_tile)])

    return k


def _finish_body(x_ref, s_ref, u_ref, dsum_ref,
                 wsrc_ref, wtgt_ref, wsd_ref, aw_ref, ab_ref,
                 btgt_ref, bd_ref, wres_ref, wmx_ref, wma_ref, bmlp_ref,
                 o_ref):
    xb = x_ref[...]
    S = s_ref[0] + s_ref[1]
    uw = u_ref[0] + u_ref[1]
    cnt = uw[:, :, R]
    dsum = dsum_ref[0] + dsum_ref[1]
    deg = jnp.sum(cnt, axis=0)[:, None]
    dtot = jnp.sum(dsum, axis=0)[:, None]

    low = jnp.zeros_like(xb)
    for t in range(T):
        u_t = uw[t, :, :R]
        y2_t = jnp.dot(xb, btgt_ref[t].T,
                       preferred_element_type=jnp.float32)
        v_t = u_t + cnt[t][:, None] * y2_t + dsum[t][:, None] * bd_ref[t][None, :]
        low = low + jnp.dot(v_t, aw_ref[t].T,
                            preferred_element_type=jnp.float32)
        low = low + cnt[t][:, None] * ab_ref[t][None, :]

    sh = jnp.dot(S, wsrc_ref[...].T, preferred_element_type=jnp.float32)
    sh = sh + deg * jnp.dot(xb, wtgt_ref[...].T,
                            preferred_element_type=jnp.float32)
    sh = sh + dtot * wsd_ref[...][None, :]

    aggr = (low + sh) / jnp.maximum(deg, 1.0)

    out = jnp.dot(xb, wres_ref[...].T, preferred_element_type=jnp.float32)
    pre = (jnp.dot(xb, wmx_ref[...].T, preferred_element_type=jnp.float32)
           + jnp.dot(aggr, wma_ref[...].T, preferred_element_type=jnp.float32)
           + bmlp_ref[...][None, :])
    o_ref[...] = out + jnp.maximum(pre, 0.0)


def _finish(x, s2, u2, dsum2, wsrc, wtgt, wsd, a_w, a_b,
            b_tgt, b_d, w_res, wm_x, wm_a, b_mlp):
    BN = 1024
    grid = (SROWS // BN,)
    return pl.pallas_call(
        _finish_body,
        grid=grid,
        in_specs=[
            pl.BlockSpec((BN, D), lambda i: (i, 0)),
            pl.BlockSpec((NC, BN, D), lambda i: (0, i, 0)),
            pl.BlockSpec((NC, T, BN, YW), lambda i: (0, 0, i, 0)),
            pl.BlockSpec((NC, T, BN), lambda i: (0, 0, i)),
            pl.BlockSpec((H, D), lambda i: (0, 0)),
            pl.BlockSpec((H, D), lambda i: (0, 0)),
            pl.BlockSpec((H,), lambda i: (0,)),
            pl.BlockSpec((T, H, R), lambda i: (0, 0, 0)),
            pl.BlockSpec((T, H), lambda i: (0, 0)),
            pl.BlockSpec((T, R, D), lambda i: (0, 0, 0)),
            pl.BlockSpec((T, R), lambda i: (0, 0)),
            pl.BlockSpec((H, D), lambda i: (0, 0)),
            pl.BlockSpec((H, D), lambda i: (0, 0)),
            pl.BlockSpec((H, H), lambda i: (0, 0)),
            pl.BlockSpec((H,), lambda i: (0,)),
        ],
        out_specs=pl.BlockSpec((BN, H), lambda i: (i, 0)),
        out_shape=jax.ShapeDtypeStruct((SROWS, H), jnp.float32),
    )(x, s2, u2, dsum2, wsrc, wtgt, wsd, a_w, a_b,
      b_tgt, b_d, w_res, wm_x, wm_a, b_mlp)


def kernel(node_embed, node_pos, edge_index, edge_attr,
           A_w, A_b, B_w, W_shared, W_res, W_mlp, b_mlp):
    E = edge_index.shape[1]
    row = edge_index[0].astype(jnp.int32)
    col = edge_index[1].astype(jnp.int32)
    etype = edge_attr[:, 0].astype(jnp.int32)
    dist = edge_attr[:, 1].astype(jnp.float32)

    wsrc = W_shared[:, :D]
    wtgt = W_shared[:, D:2 * D]
    wsd = W_shared[:, 2 * D]
    b_src = B_w[:, :, :D]
    b_tgt = B_w[:, :, D:2 * D]
    b_d = B_w[:, :, 2 * D]
    wm_x = W_mlp[:, :D]
    wm_a = W_mlp[:, D:]

    def padded(arr, fill, chunk):
        per_tile = 2 * NW * chunk
        ep = ((E + per_tile - 1) // per_tile) * per_tile
        nch = ep // (NW * chunk)
        p = jnp.concatenate(
            [arr, jnp.full((ep - E,), fill, arr.dtype)])
        return p.reshape(NW, nch, chunk), nch

    gsrc = etype * N + row
    gdst = etype * SROWS + col
    row_p, nch_s = padded(row, 0, CHUNK_S)
    col_p, _ = padded(col, N, CHUNK_S)
    gsrc_p, nch_u = padded(gsrc, 0, CHUNK)
    gdst_p, _ = padded(gdst, UROWS - 1, CHUNK)
    dist_p, _ = padded(dist, 0.0, CHUNK)

    ytab = _build_y1(node_embed, b_src).reshape(T * N, YW)

    (s2,) = _sc_s_kernel(nch_s)(node_embed, row_p, col_p)
    u2, dsum2 = _sc_u_kernel(nch_u)(ytab, gsrc_p, gdst_p, dist_p)

    u2 = u2.reshape(NC, T, SROWS, YW)
    dsum2 = dsum2.reshape(NC, T, SROWS)

    x_pad = jnp.concatenate(
        [node_embed, jnp.zeros((SROWS - N, D), jnp.float32)])
    out = _finish(x_pad, s2, u2, dsum2, wsrc, wtgt, wsd,
                  A_w, A_b, b_tgt, b_d, W_res, wm_x, wm_a, b_mlp)
    return out[:N]


# speedup vs baseline: 7.7103x; 1.2271x over previous
import functools

import jax
import jax.numpy as jnp
from jax import lax
from jax.experimental import pallas as pl
from jax.experimental.pallas import tpu as pltpu
from jax.experimental.pallas import tpu_sc as plsc

N = 10000
D = 128
H = 128
R = 16
T = 4

NC = 2
NS = 16
NW = NC * NS
CHUNK = 128
CHUNK_S = 64
SCOPY = 64

NCH_S0, NCH_S1 = 236, 80
NCH_U0, NCH_U1 = 92, 68

SROWS = 10240
UROWS = 4 * SROWS


YW = 24


def _y1_body(x_ref, b_ref, o_ref):
    y = jnp.dot(x_ref[...], b_ref[0].T, preferred_element_type=jnp.float32)
    one = jnp.ones((N, 1), jnp.float32)
    zero = jnp.zeros((N, YW - R - 1), jnp.float32)
    o_ref[0] = jnp.concatenate([y, one, zero], axis=-1)


def _build_y1(x, b_src):
    return pl.pallas_call(
        _y1_body,
        grid=(T,),
        in_specs=[
            pl.BlockSpec((N, D), lambda t: (0, 0)),
            pl.BlockSpec((1, R, D), lambda t: (t, 0, 0)),
        ],
        out_specs=pl.BlockSpec((1, N, YW), lambda t: (t, 0, 0)),
        out_shape=jax.ShapeDtypeStruct((T, N, YW), jnp.float32),
    )(x, b_src)


_SC_MESH = dict(core_axis_name="c", subcore_axis_name="s")


def _sc_s_kernel(nch_max):
    @functools.partial(
        pl.kernel,
        mesh=plsc.VectorSubcoreMesh(**_SC_MESH),
        compiler_params=pltpu.CompilerParams(use_tc_tiling_on_sc=False),
        out_type=[
            jax.ShapeDtypeStruct((NC, SROWS, D), jnp.float32),
        ],
        scratch_types=[
            pltpu.VMEM((nch_max, CHUNK_S), jnp.int32),
            pltpu.VMEM((nch_max, CHUNK_S), jnp.int32),
            pltpu.VMEM((CHUNK_S, D), jnp.float32),
            pltpu.VMEM((CHUNK_S, D), jnp.float32),
            pltpu.VMEM_SHARED((SROWS, D), jnp.float32),
            pltpu.SemaphoreType.DMA,
            pltpu.SemaphoreType.DMA,
        ],
    )
    def k(x_hbm, row_h, col_h, s_out, row3, col3, xb0, xb1, s_sh, sm0, sm1):
        cid = lax.axis_index("c")
        sid = lax.axis_index("s")
        gid = cid * NS + sid
        n_pairs = jnp.where(cid == 0, NCH_S0 // 2, NCH_S1 // 2)

        pltpu.sync_copy(row_h.at[gid], row3)
        pltpu.sync_copy(col_h.at[gid], col3)

        zx = jnp.zeros((16,), jnp.float32)
        def zxb(i, _):
            def zrow(j, _):
                xb0[i, pl.ds(j * 16, 16)] = zx
                return 0
            lax.fori_loop(0, D // 16, zrow, 0)
            return 0
        lax.fori_loop(0, CHUNK_S, zxb, 0)

        s_per_tile = SROWS // NS
        def zs(i, _):
            base = sid * s_per_tile + i * SCOPY
            pltpu.sync_copy(xb0.at[pl.ds(0, SCOPY)],
                            s_sh.at[pl.ds(base, SCOPY)])
            return 0
        lax.fori_loop(0, s_per_tile // SCOPY, zs, 0)

        plsc.subcore_barrier()

        pltpu.async_copy(x_hbm.at[row3.at[0]], xb0, sm0)

        def pair(p, _):
            @pl.when(p < n_pairs)
            def _():
                c0 = 2 * p
                pltpu.async_copy(x_hbm.at[row3.at[c0 + 1]], xb1, sm1)
                pltpu.make_async_copy(x_hbm.at[row3.at[c0]], xb0, sm0).wait()
                pltpu.sync_copy(xb0, s_sh.at[col3.at[c0]], add=True)

                @pl.when(p < n_pairs - 1)
                def _():
                    pltpu.async_copy(x_hbm.at[row3.at[c0 + 2]], xb0, sm0)
                pltpu.make_async_copy(
                    x_hbm.at[row3.at[c0 + 1]], xb1, sm1).wait()
                pltpu.sync_copy(xb1, s_sh.at[col3.at[c0 + 1]], add=True)
            return 0

        lax.fori_loop(0, max(NCH_S0, NCH_S1) // 2, pair, 0)

        plsc.subcore_barrier()

        def ws(i, _):
            base = sid * s_per_tile + i * SCOPY
            pltpu.sync_copy(s_sh.at[pl.ds(base, SCOPY)],
                            s_out.at[cid, pl.ds(base, SCOPY)])
            return 0
        lax.fori_loop(0, s_per_tile // SCOPY, ws, 0)

    return k


def _sc_u_kernel(nch_max):
    @functools.partial(
        pl.kernel,
        mesh=plsc.VectorSubcoreMesh(**_SC_MESH),
        compiler_params=pltpu.CompilerParams(use_tc_tiling_on_sc=False),
        out_type=[
            jax.ShapeDtypeStruct((NC, UROWS, YW), jnp.float32),
            jax.ShapeDtypeStruct((NC, UROWS), jnp.float32),
        ],
        scratch_types=[
            pltpu.VMEM((nch_max, CHUNK), jnp.int32),
            pltpu.VMEM((nch_max, CHUNK), jnp.int32),
            pltpu.VMEM((nch_max, CHUNK), jnp.float32),
            pltpu.VMEM((CHUNK, YW), jnp.float32),
            pltpu.VMEM((CHUNK, YW), jnp.float32),
            pltpu.VMEM_SHARED((UROWS, YW), jnp.float32),
            pltpu.VMEM_SHARED((UROWS,), jnp.float32),
            pltpu.SemaphoreType.DMA,
            pltpu.SemaphoreType.DMA,
        ],
    )
    def k(y1_hbm, gsrc_h, gdst_h, dist_h, u_out, dsum_out,
          gsrc3, gdst3, dist3, yb0, yb1, u_sh, dsum_sh, sm0, sm1):
        cid = lax.axis_index("c")
        sid = lax.axis_index("s")
        gid = cid * NS + sid
        n_pairs = jnp.where(cid == 0, NCH_U0 // 2, NCH_U1 // 2)

        pltpu.sync_copy(gsrc_h.at[gid], gsrc3)
        pltpu.sync_copy(gdst_h.at[gid], gdst3)
        pltpu.sync_copy(dist_h.at[gid], dist3)

        zx = jnp.zeros((16,), jnp.float32)
        def zyb(i, _):
            yb0[i, pl.ds(0, 16)] = zx
            yb0[i, pl.ds(YW - 16, 16)] = zx
            return 0
        lax.fori_loop(0, CHUNK, zyb, 0)
        def zd(j, _):
            dist3[0, pl.ds(j * 16, 16)] = zx
            return 0
        lax.fori_loop(0, CHUNK // 16, zd, 0)

        u_per_tile = UROWS // NS
        def zu(i, _):
            base = sid * u_per_tile + i * CHUNK
            pltpu.sync_copy(yb0, u_sh.at[pl.ds(base, CHUNK)])
            pltpu.sync_copy(dist3.at[0], dsum_sh.at[pl.ds(base, CHUNK)])
            return 0
        lax.fori_loop(0, u_per_tile // CHUNK, zu, 0)

        plsc.subcore_barrier()

        pltpu.sync_copy(dist_h.at[gid], dist3)
        pltpu.async_copy(y1_hbm.at[gsrc3.at[0]], yb0, sm0)

        def pair(p, _):
            @pl.when(p < n_pairs)
            def _():
                c0 = 2 * p
                pltpu.async_copy(y1_hbm.at[gsrc3.at[c0 + 1]], yb1, sm1)
                pltpu.make_async_copy(
                    y1_hbm.at[gsrc3.at[c0]], yb0, sm0).wait()
                pltpu.sync_copy(yb0, u_sh.at[gdst3.at[c0]], add=True)
                pltpu.sync_copy(dist3.at[c0],
                                dsum_sh.at[gdst3.at[c0]], add=True)

                @pl.when(p < n_pairs - 1)
                def _():
                    pltpu.async_copy(y1_hbm.at[gsrc3.at[c0 + 2]], yb0, sm0)
                pltpu.make_async_copy(
                    y1_hbm.at[gsrc3.at[c0 + 1]], yb1, sm1).wait()
                pltpu.sync_copy(yb1, u_sh.at[gdst3.at[c0 + 1]], add=True)
                pltpu.sync_copy(dist3.at[c0 + 1],
                                dsum_sh.at[gdst3.at[c0 + 1]], add=True)
            return 0

        lax.fori_loop(0, max(NCH_U0, NCH_U1) // 2, pair, 0)

        plsc.subcore_barrier()

        def wu(i, _):
            base = sid * u_per_tile + i * CHUNK
            pltpu.sync_copy(u_sh.at[pl.ds(base, CHUNK)],
                            u_out.at[cid, pl.ds(base, CHUNK)])
            return 0
        lax.fori_loop(0, u_per_tile // CHUNK, wu, 0)
        ubase = sid * u_per_tile
        pltpu.sync_copy(dsum_sh.at[pl.ds(ubase, u_per_tile)],
                        dsum_out.at[cid, pl.ds(ubase, u_per_tile)])

    return k


def _finish_body(x_ref, s_ref, u_ref, dsum_ref,
                 wsrc_ref, wtgt_ref, wsd_ref, aw_ref, ab_ref,
                 btgt_ref, bd_ref, wres_ref, wmx_ref, wma_ref, bmlp_ref,
                 o_ref):
    xb = x_ref[...]
    S = s_ref[0] + s_ref[1]
    uw = u_ref[0] + u_ref[1]
    cnt = uw[:, :, R]
    dsum = dsum_ref[0] + dsum_ref[1]
    deg = jnp.sum(cnt, axis=0)[:, None]
    dtot = jnp.sum(dsum, axis=0)[:, None]

    low = jnp.zeros_like(xb)
    for t in range(T):
        u_t = uw[t, :, :R]
        y2_t = jnp.dot(xb, btgt_ref[t].T,
                       preferred_element_type=jnp.float32)
        v_t = u_t + cnt[t][:, None] * y2_t + dsum[t][:, None] * bd_ref[t][None, :]
        low = low + jnp.dot(v_t, aw_ref[t].T,
                            preferred_element_type=jnp.float32)
        low = low + cnt[t][:, None] * ab_ref[t][None, :]

    sh = jnp.dot(S, wsrc_ref[...].T, preferred_element_type=jnp.float32)
    sh = sh + deg * jnp.dot(xb, wtgt_ref[...].T,
                            preferred_element_type=jnp.float32)
    sh = sh + dtot * wsd_ref[...][None, :]

    aggr = (low + sh) / jnp.maximum(deg, 1.0)

    out = jnp.dot(xb, wres_ref[...].T, preferred_element_type=jnp.float32)
    pre = (jnp.dot(xb, wmx_ref[...].T, preferred_element_type=jnp.float32)
           + jnp.dot(aggr, wma_ref[...].T, preferred_element_type=jnp.float32)
           + bmlp_ref[...][None, :])
    o_ref[...] = out + jnp.maximum(pre, 0.0)


def _finish(x, s2, u2, dsum2, wsrc, wtgt, wsd, a_w, a_b,
            b_tgt, b_d, w_res, wm_x, wm_a, b_mlp):
    BN = 1024
    grid = (SROWS // BN,)
    return pl.pallas_call(
        _finish_body,
        grid=grid,
        in_specs=[
            pl.BlockSpec((BN, D), lambda i: (i, 0)),
            pl.BlockSpec((NC, BN, D), lambda i: (0, i, 0)),
            pl.BlockSpec((NC, T, BN, YW), lambda i: (0, 0, i, 0)),
            pl.BlockSpec((NC, T, BN), lambda i: (0, 0, i)),
            pl.BlockSpec((H, D), lambda i: (0, 0)),
            pl.BlockSpec((H, D), lambda i: (0, 0)),
            pl.BlockSpec((H,), lambda i: (0,)),
            pl.BlockSpec((T, H, R), lambda i: (0, 0, 0)),
            pl.BlockSpec((T, H), lambda i: (0, 0)),
            pl.BlockSpec((T, R, D), lambda i: (0, 0, 0)),
            pl.BlockSpec((T, R), lambda i: (0, 0)),
            pl.BlockSpec((H, D), lambda i: (0, 0)),
            pl.BlockSpec((H, D), lambda i: (0, 0)),
            pl.BlockSpec((H, H), lambda i: (0, 0)),
            pl.BlockSpec((H,), lambda i: (0,)),
        ],
        out_specs=pl.BlockSpec((BN, H), lambda i: (i, 0)),
        out_shape=jax.ShapeDtypeStruct((SROWS, H), jnp.float32),
    )(x, s2, u2, dsum2, wsrc, wtgt, wsd, a_w, a_b,
      b_tgt, b_d, w_res, wm_x, wm_a, b_mlp)


def kernel(node_embed, node_pos, edge_index, edge_attr,
           A_w, A_b, B_w, W_shared, W_res, W_mlp, b_mlp):
    E = edge_index.shape[1]
    row = edge_index[0].astype(jnp.int32)
    col = edge_index[1].astype(jnp.int32)
    etype = edge_attr[:, 0].astype(jnp.int32)
    dist = edge_attr[:, 1].astype(jnp.float32)

    wsrc = W_shared[:, :D]
    wtgt = W_shared[:, D:2 * D]
    wsd = W_shared[:, 2 * D]
    b_src = B_w[:, :, :D]
    b_tgt = B_w[:, :, D:2 * D]
    b_d = B_w[:, :, 2 * D]
    wm_x = W_mlp[:, :D]
    wm_a = W_mlp[:, D:]

    def layout(arr, fill, chunk, nch0, nch1):
        tot0 = NS * nch0 * chunk
        tot1 = NS * nch1 * chunk
        p = jnp.concatenate(
            [arr, jnp.full((tot0 + tot1 - E,), fill, arr.dtype)])
        nch_max = max(nch0, nch1)
        a0 = p[:tot0].reshape(NS, nch0, chunk)
        a1 = p[tot0:].reshape(NS, nch1, chunk)
        a0 = jnp.concatenate(
            [a0, jnp.full((NS, nch_max - nch0, chunk), fill, arr.dtype)], 1)
        a1 = jnp.concatenate(
            [a1, jnp.full((NS, nch_max - nch1, chunk), fill, arr.dtype)], 1)
        return jnp.concatenate([a0, a1], 0)

    gsrc = etype * N + row
    gdst = etype * SROWS + col
    row_p = layout(row, 0, CHUNK_S, NCH_S0, NCH_S1)
    col_p = layout(col, N, CHUNK_S, NCH_S0, NCH_S1)
    gsrc_p = layout(gsrc, 0, CHUNK, NCH_U0, NCH_U1)
    gdst_p = layout(gdst, UROWS - 1, CHUNK, NCH_U0, NCH_U1)
    dist_p = layout(dist, 0.0, CHUNK, NCH_U0, NCH_U1)

    ytab = _build_y1(node_embed, b_src).reshape(T * N, YW)

    (s2,) = _sc_s_kernel(max(NCH_S0, NCH_S1))(node_embed, row_p, col_p)
    u2, dsum2 = _sc_u_kernel(max(NCH_U0, NCH_U1))(ytab, gsrc_p, gdst_p, dist_p)

    u2 = u2.reshape(NC, T, SROWS, YW)
    dsum2 = dsum2.reshape(NC, T, SROWS)

    x_pad = jnp.concatenate(
        [node_embed, jnp.zeros((SROWS - N, D), jnp.float32)])
    out = _finish(x_pad, s2, u2, dsum2, wsrc, wtgt, wsd,
                  A_w, A_b, b_tgt, b_d, W_res, wm_x, wm_a, b_mlp)
    return out[:N]
